# baseline (speedup 1.0000x reference)
import functools

import jax
import jax.numpy as jnp
from jax import lax
from jax.experimental import pallas as pl
from jax.experimental.pallas import tpu as pltpu
from jax.experimental.pallas import tpu_sc as plsc

NC = 2
NS = 16
NL = 16
NW = NC * NS

CH = 80


def _node_mm_body(h_ref, wst_ref, wdt_ref, p_ref, q_ref):
    hb = h_ref[...]
    p_ref[...] = jnp.dot(
        hb, wst_ref[...], preferred_element_type=jnp.float32)
    q_ref[...] = jnp.dot(
        hb, wdt_ref[...], preferred_element_type=jnp.float32)


def _node_mm(h, wst, wdt, block_rows):
    n, d = h.shape
    d2 = wst.shape[1]
    grid = (n // block_rows,)
    return pl.pallas_call(
        _node_mm_body,
        grid=grid,
        in_specs=[
            pl.BlockSpec((block_rows, d), lambda i: (i, 0)),
            pl.BlockSpec((d, d2), lambda i: (0, 0)),
            pl.BlockSpec((d, d2), lambda i: (0, 0)),
        ],
        out_specs=[
            pl.BlockSpec((block_rows, d2), lambda i: (i, 0)),
            pl.BlockSpec((block_rows, d2), lambda i: (i, 0)),
        ],
        out_shape=[
            jax.ShapeDtypeStruct((n, d2), jnp.float32),
            jax.ShapeDtypeStruct((n, d2), jnp.float32),
        ],
    )(h, wst, wdt)


def _sc_gather(p, q, src, dst):
    n, d2 = p.shape
    e = src.shape[0]
    epw = e // NW
    nch = epw // CH
    mesh = plsc.VectorSubcoreMesh(core_axis_name="c", subcore_axis_name="s")

    @functools.partial(
        pl.kernel,
        out_type=[
            jax.ShapeDtypeStruct((e, d2), jnp.float32),
            jax.ShapeDtypeStruct((e, d2), jnp.float32),
        ],
        mesh=mesh,
        scratch_types=[
            pltpu.VMEM((CH,), jnp.int32),
            pltpu.VMEM((CH,), jnp.int32),
            pltpu.VMEM((CH, d2), jnp.float32),
            pltpu.VMEM((CH, d2), jnp.float32),
            pltpu.SemaphoreType.DMA,
            pltpu.SemaphoreType.DMA,
        ],
    )
    def gather_k(p_hbm, q_hbm, src_hbm, dst_hbm, z1_hbm, z2_hbm,
                 sidx, didx, pbuf, qbuf, sem_p, sem_q):
        wid = lax.axis_index("s") * NC + lax.axis_index("c")
        base0 = wid * epw

        def chunk_body(i, carry):
            base = base0 + i * CH
            pltpu.sync_copy(src_hbm.at[pl.ds(base, CH)], sidx)
            pltpu.sync_copy(dst_hbm.at[pl.ds(base, CH)], didx)
            cp_p = pltpu.async_copy(p_hbm.at[sidx], pbuf, sem_p)
            cp_q = pltpu.async_copy(q_hbm.at[didx], qbuf, sem_q)
            cp_p.wait()
            cp_q.wait()
            pltpu.sync_copy(pbuf, z1_hbm.at[pl.ds(base, CH)])
            pltpu.sync_copy(qbuf, z2_hbm.at[pl.ds(base, CH)])
            return carry

        lax.fori_loop(0, nch, chunk_body, 0)

    return gather_k(p, q, src, dst)


def _edge_mm_body(z1_ref, z2_ref, ef_ref, wet_ref, bgc_ref, m_ref):
    d = m_ref.shape[1]
    t = z1_ref[...].astype(jnp.float32) + z2_ref[...].astype(
        jnp.float32) + jnp.dot(
        ef_ref[...], wet_ref[...],
        preferred_element_type=jnp.float32) + bgc_ref[...]
    zg = t[:, :d]
    zc = t[:, d:]
    m_ref[...] = jax.nn.sigmoid(zg) * jax.nn.softplus(zc)


def _edge_mm(z1, z2, ef, wet, bgc, block_rows):
    e, d2 = z1.shape
    de = ef.shape[1]
    d = d2 // 2
    grid = (e // block_rows,)
    return pl.pallas_call(
        _edge_mm_body,
        grid=grid,
        in_specs=[
            pl.BlockSpec((block_rows, d2), lambda i: (i, 0)),
            pl.BlockSpec((block_rows, d2), lambda i: (i, 0)),
            pl.BlockSpec((block_rows, de), lambda i: (i, 0)),
            pl.BlockSpec((de, d2), lambda i: (0, 0)),
            pl.BlockSpec((1, d2), lambda i: (0, 0)),
        ],
        out_specs=pl.BlockSpec((block_rows, d), lambda i: (i, 0)),
        out_shape=jax.ShapeDtypeStruct((e, d), jnp.float32),
    )(z1, z2, ef, wet, bgc)


def _sc_scatter(m, src, n):
    e, d = m.shape
    epw = e // NW
    nch = epw // CH
    fc = 128
    npt = ((n + NS * fc - 1) // (NS * fc)) * fc
    n_pad = npt * NS
    nfc = npt // fc
    mesh = plsc.VectorSubcoreMesh(core_axis_name="c", subcore_axis_name="s")

    @functools.partial(
        pl.kernel,
        out_type=jax.ShapeDtypeStruct((NC, n_pad, d), jnp.float32),
        mesh=mesh,
        scratch_types=[
            pltpu.VMEM((CH,), jnp.int32),
            pltpu.VMEM((CH, d), jnp.float32),
            pltpu.VMEM((fc, d), jnp.float32),
            pltpu.VMEM_SHARED((n_pad, d), jnp.float32),
        ],
    )
    def scatter_k(m_hbm, src_hbm, out_hbm, idxv, mbuf, stg, acc_sh):
        cid = lax.axis_index("c")
        sid = lax.axis_index("s")
        wid = sid * NC + cid
        base0 = wid * epw

        def zero_body(r, c):
            for k in range(d // NL):
                stg[r, pl.ds(k * NL, NL)] = jnp.zeros((NL,), jnp.float32)
            return c

        lax.fori_loop(0, fc, zero_body, 0)
        for j in range(nfc):
            pltpu.sync_copy(stg, acc_sh.at[pl.ds(sid * npt + j * fc, fc)])
        plsc.subcore_barrier()

        def chunk_body(i, c):
            base = base0 + i * CH
            pltpu.sync_copy(src_hbm.at[pl.ds(base, CH)], idxv)
            pltpu.sync_copy(m_hbm.at[pl.ds(base, CH)], mbuf)
            pltpu.sync_copy(mbuf, acc_sh.at[idxv], add=True)
            return c

        lax.fori_loop(0, nch, chunk_body, 0)
        plsc.subcore_barrier()
        for j in range(nfc):
            r0 = sid * npt + j * fc
            pltpu.sync_copy(acc_sh.at[pl.ds(r0, fc)], stg)
            pltpu.sync_copy(stg, out_hbm.at[cid, pl.ds(r0, fc)])

    return scatter_k(m, src)


def _bn_body(a0_ref, a1_ref, h_ref, gamma_ref, beta_ref, o_ref):
    agg = a0_ref[...] + a1_ref[...]
    n = agg.shape[0]
    mean = jnp.sum(agg, axis=0, keepdims=True) / n
    cen = agg - mean
    var = jnp.sum(cen * cen, axis=0, keepdims=True) / n
    xb = cen * lax.rsqrt(var + 1e-5) * gamma_ref[...] + beta_ref[...]
    o_ref[...] = jax.nn.softplus(h_ref[...] + xb)


def _bn_final(a0, a1, h, gamma, beta):
    n, d = h.shape
    return pl.pallas_call(
        _bn_body,
        out_shape=jax.ShapeDtypeStruct((n, d), jnp.float32),
    )(a0, a1, h, gamma, beta)


def kernel(h, edge_index, edge_feat, Wg, bg, Wc, bc, gamma, beta):
    n, d = h.shape
    e = edge_index.shape[1]
    src = edge_index[0]
    dst = edge_index[1]

    wst = jnp.concatenate([Wg[:, :d], Wc[:, :d]], axis=0).T
    wdt = jnp.concatenate([Wg[:, d:2 * d], Wc[:, d:2 * d]], axis=0).T
    wet = jnp.concatenate([Wg[:, 2 * d:], Wc[:, 2 * d:]], axis=0).T
    bgc = jnp.concatenate([bg, bc]).reshape(1, 2 * d)

    p, q = _node_mm(h, wst, wdt, block_rows=2000)
    z1, z2 = _sc_gather(p, q, src, dst)
    m = _edge_mm(z1, z2, edge_feat, wet, bgc, block_rows=3200)
    parts = _sc_scatter(m, src, n)
    return _bn_final(parts[0, :n], parts[1, :n], h,
                     gamma.reshape(1, d), beta.reshape(1, d))


# speedup vs baseline: 1.2660x; 1.2660x over previous
import functools

import jax
import jax.numpy as jnp
from jax import lax
from jax.experimental import pallas as pl
from jax.experimental.pallas import tpu as pltpu
from jax.experimental.pallas import tpu_sc as plsc

NC = 2
NS = 16
NL = 16
NW = NC * NS

CH = 80


def _pack_bf16(a, b):
    ai = lax.bitcast_convert_type(a, jnp.int32)
    bi = lax.bitcast_convert_type(b, jnp.int32)
    ar = ai + jnp.int32(0x7FFF) + ((ai >> 16) & 1)
    br = bi + jnp.int32(0x7FFF) + ((bi >> 16) & 1)
    lo = (ar >> 16) & jnp.int32(0xFFFF)
    hi = br & jnp.int32(-0x10000)
    return lo | hi


def _unpack_bf16(x):
    lo = lax.bitcast_convert_type(x << 16, jnp.float32)
    hi = lax.bitcast_convert_type(x & jnp.int32(-0x10000), jnp.float32)
    return lo, hi


def _node_mm_body(h_ref, wsg_ref, wsc_ref, wdg_ref, wdc_ref, p_ref, q_ref):
    hb = h_ref[...]
    pg = jnp.dot(hb, wsg_ref[...], preferred_element_type=jnp.float32)
    pc = jnp.dot(hb, wsc_ref[...], preferred_element_type=jnp.float32)
    qg = jnp.dot(hb, wdg_ref[...], preferred_element_type=jnp.float32)
    qc = jnp.dot(hb, wdc_ref[...], preferred_element_type=jnp.float32)
    p_ref[...] = _pack_bf16(pg, pc)
    q_ref[...] = _pack_bf16(qg, qc)


def _node_mm(h, wsg, wsc, wdg, wdc, block_rows):
    n, d = h.shape
    grid = (n // block_rows,)
    wspec = pl.BlockSpec((d, d), lambda i: (0, 0))
    return pl.pallas_call(
        _node_mm_body,
        grid=grid,
        in_specs=[
            pl.BlockSpec((block_rows, d), lambda i: (i, 0)),
            wspec, wspec, wspec, wspec,
        ],
        out_specs=[
            pl.BlockSpec((block_rows, d), lambda i: (i, 0)),
            pl.BlockSpec((block_rows, d), lambda i: (i, 0)),
        ],
        out_shape=[
            jax.ShapeDtypeStruct((n, d), jnp.int32),
            jax.ShapeDtypeStruct((n, d), jnp.int32),
        ],
    )(h, wsg, wsc, wdg, wdc)


def _sc_gather(p, q, src, dst):
    n, d = p.shape
    e = src.shape[0]
    epw = e // NW
    nch = epw // CH
    mesh = plsc.VectorSubcoreMesh(core_axis_name="c", subcore_axis_name="s")

    @functools.partial(
        pl.kernel,
        out_type=[
            jax.ShapeDtypeStruct((e, d), jnp.int32),
            jax.ShapeDtypeStruct((e, d), jnp.int32),
        ],
        mesh=mesh,
        scratch_types=[
            pltpu.VMEM((CH,), jnp.int32),
            pltpu.VMEM((CH,), jnp.int32),
            pltpu.VMEM((CH, d), jnp.int32),
            pltpu.VMEM((CH, d), jnp.int32),
            pltpu.SemaphoreType.DMA,
            pltpu.SemaphoreType.DMA,
        ],
    )
    def gather_k(p_hbm, q_hbm, src_hbm, dst_hbm, z1_hbm, z2_hbm,
                 sidx, didx, pbuf, qbuf, sem_p, sem_q):
        wid = lax.axis_index("s") * NC + lax.axis_index("c")
        base0 = wid * epw

        def chunk_body(i, carry):
            base = base0 + i * CH
            pltpu.sync_copy(src_hbm.at[pl.ds(base, CH)], sidx)
            pltpu.sync_copy(dst_hbm.at[pl.ds(base, CH)], didx)
            cp_p = pltpu.async_copy(p_hbm.at[sidx], pbuf, sem_p)
            cp_q = pltpu.async_copy(q_hbm.at[didx], qbuf, sem_q)
            cp_p.wait()
            cp_q.wait()
            pltpu.sync_copy(pbuf, z1_hbm.at[pl.ds(base, CH)])
            pltpu.sync_copy(qbuf, z2_hbm.at[pl.ds(base, CH)])
            return carry

        lax.fori_loop(0, nch, chunk_body, 0)

    return gather_k(p, q, src, dst)


def _edge_mm_body(z1_ref, z2_ref, ef_ref, wet_ref, bgc_ref, m_ref):
    d = m_ref.shape[1]
    r = jnp.dot(ef_ref[...], wet_ref[...],
                preferred_element_type=jnp.float32) + bgc_ref[...]
    g1, c1 = _unpack_bf16(z1_ref[...])
    g2, c2 = _unpack_bf16(z2_ref[...])
    zg = g1 + g2 + r[:, :d]
    zc = c1 + c2 + r[:, d:]
    m_ref[...] = jax.nn.sigmoid(zg) * jax.nn.softplus(zc)


def _edge_mm(z1, z2, ef, wet, bgc, block_rows):
    e, d = z1.shape
    de = ef.shape[1]
    d2 = 2 * d
    grid = (e // block_rows,)
    return pl.pallas_call(
        _edge_mm_body,
        grid=grid,
        in_specs=[
            pl.BlockSpec((block_rows, d), lambda i: (i, 0)),
            pl.BlockSpec((block_rows, d), lambda i: (i, 0)),
            pl.BlockSpec((block_rows, de), lambda i: (i, 0)),
            pl.BlockSpec((de, d2), lambda i: (0, 0)),
            pl.BlockSpec((1, d2), lambda i: (0, 0)),
        ],
        out_specs=pl.BlockSpec((block_rows, d), lambda i: (i, 0)),
        out_shape=jax.ShapeDtypeStruct((e, d), jnp.float32),
    )(z1, z2, ef, wet, bgc)


def _sc_scatter(m, src, n):
    e, d = m.shape
    epw = e // NW
    nch = epw // CH
    fc = 128
    npt = ((n + NS * fc - 1) // (NS * fc)) * fc
    n_pad = npt * NS
    nfc = npt // fc
    mesh = plsc.VectorSubcoreMesh(core_axis_name="c", subcore_axis_name="s")

    @functools.partial(
        pl.kernel,
        out_type=jax.ShapeDtypeStruct((NC, n_pad, d), jnp.float32),
        mesh=mesh,
        scratch_types=[
            pltpu.VMEM((CH,), jnp.int32),
            pltpu.VMEM((CH, d), jnp.float32),
            pltpu.VMEM((fc, d), jnp.float32),
            pltpu.VMEM_SHARED((n_pad, d), jnp.float32),
        ],
    )
    def scatter_k(m_hbm, src_hbm, out_hbm, idxv, mbuf, stg, acc_sh):
        cid = lax.axis_index("c")
        sid = lax.axis_index("s")
        wid = sid * NC + cid
        base0 = wid * epw

        def zero_body(r, c):
            for k in range(d // NL):
                stg[r, pl.ds(k * NL, NL)] = jnp.zeros((NL,), jnp.float32)
            return c

        lax.fori_loop(0, fc, zero_body, 0)
        for j in range(nfc):
            pltpu.sync_copy(stg, acc_sh.at[pl.ds(sid * npt + j * fc, fc)])
        plsc.subcore_barrier()

        def chunk_body(i, c):
            base = base0 + i * CH
            pltpu.sync_copy(src_hbm.at[pl.ds(base, CH)], idxv)
            pltpu.sync_copy(m_hbm.at[pl.ds(base, CH)], mbuf)
            pltpu.sync_copy(mbuf, acc_sh.at[idxv], add=True)
            return c

        lax.fori_loop(0, nch, chunk_body, 0)
        plsc.subcore_barrier()
        for j in range(nfc):
            r0 = sid * npt + j * fc
            pltpu.sync_copy(acc_sh.at[pl.ds(r0, fc)], stg)
            pltpu.sync_copy(stg, out_hbm.at[cid, pl.ds(r0, fc)])

    return scatter_k(m, src)


def _bn_body(a0_ref, a1_ref, h_ref, gamma_ref, beta_ref, o_ref):
    agg = a0_ref[...] + a1_ref[...]
    n = agg.shape[0]
    mean = jnp.sum(agg, axis=0, keepdims=True) / n
    cen = agg - mean
    var = jnp.sum(cen * cen, axis=0, keepdims=True) / n
    xb = cen * lax.rsqrt(var + 1e-5) * gamma_ref[...] + beta_ref[...]
    o_ref[...] = jax.nn.softplus(h_ref[...] + xb)


def _bn_final(a0, a1, h, gamma, beta):
    n, d = h.shape
    return pl.pallas_call(
        _bn_body,
        out_shape=jax.ShapeDtypeStruct((n, d), jnp.float32),
    )(a0, a1, h, gamma, beta)


def kernel(h, edge_index, edge_feat, Wg, bg, Wc, bc, gamma, beta):
    n, d = h.shape
    src = edge_index[0]
    dst = edge_index[1]

    wsg = Wg[:, :d].T
    wsc = Wc[:, :d].T
    wdg = Wg[:, d:2 * d].T
    wdc = Wc[:, d:2 * d].T
    wet = jnp.concatenate([Wg[:, 2 * d:], Wc[:, 2 * d:]], axis=0).T
    bgc = jnp.concatenate([bg, bc]).reshape(1, 2 * d)

    p, q = _node_mm(h, wsg, wsc, wdg, wdc, block_rows=2000)
    z1, z2 = _sc_gather(p, q, src, dst)
    m = _edge_mm(z1, z2, edge_feat, wet, bgc, block_rows=3200)
    parts = _sc_scatter(m, src, n)
    return _bn_final(parts[0, :n], parts[1, :n], h,
                     gamma.reshape(1, d), beta.reshape(1, d))


# speedup vs baseline: 1.3247x; 1.0464x over previous
import functools

import jax
import jax.numpy as jnp
from jax import lax
from jax.experimental import pallas as pl
from jax.experimental.pallas import tpu as pltpu
from jax.experimental.pallas import tpu_sc as plsc

NC = 2
NS = 16
NL = 16
NW = NC * NS

CH = 80


def _pack_bf16(a, b):
    ai = lax.bitcast_convert_type(a, jnp.int32)
    bi = lax.bitcast_convert_type(b, jnp.int32)
    ar = ai + jnp.int32(0x7FFF) + ((ai >> 16) & 1)
    br = bi + jnp.int32(0x7FFF) + ((bi >> 16) & 1)
    lo = (ar >> 16) & jnp.int32(0xFFFF)
    hi = br & jnp.int32(-0x10000)
    return lo | hi


def _unpack_bf16(x):
    lo = lax.bitcast_convert_type(x << 16, jnp.float32)
    hi = lax.bitcast_convert_type(x & jnp.int32(-0x10000), jnp.float32)
    return lo, hi


def _node_mm_body(h_ref, wsg_ref, wsc_ref, wdg_ref, wdc_ref, p_ref, q_ref):
    hb = h_ref[...]
    pg = jnp.dot(hb, wsg_ref[...], preferred_element_type=jnp.float32)
    pc = jnp.dot(hb, wsc_ref[...], preferred_element_type=jnp.float32)
    qg = jnp.dot(hb, wdg_ref[...], preferred_element_type=jnp.float32)
    qc = jnp.dot(hb, wdc_ref[...], preferred_element_type=jnp.float32)
    p_ref[...] = _pack_bf16(pg, pc)
    q_ref[...] = _pack_bf16(qg, qc)


def _node_mm(h, wsg, wsc, wdg, wdc, block_rows):
    n, d = h.shape
    grid = (n // block_rows,)
    wspec = pl.BlockSpec((d, d), lambda i: (0, 0))
    return pl.pallas_call(
        _node_mm_body,
        grid=grid,
        in_specs=[
            pl.BlockSpec((block_rows, d), lambda i: (i, 0)),
            wspec, wspec, wspec, wspec,
        ],
        out_specs=[
            pl.BlockSpec((block_rows, d), lambda i: (i, 0)),
            pl.BlockSpec((block_rows, d), lambda i: (i, 0)),
        ],
        out_shape=[
            jax.ShapeDtypeStruct((n, d), jnp.int32),
            jax.ShapeDtypeStruct((n, d), jnp.int32),
        ],
    )(h, wsg, wsc, wdg, wdc)


def _sc_gather(p, q, src, dst):
    n, d = p.shape
    e = src.shape[0]
    epw = e // NW
    nch = epw // CH
    mesh = plsc.VectorSubcoreMesh(core_axis_name="c", subcore_axis_name="s")

    @functools.partial(
        pl.kernel,
        out_type=[
            jax.ShapeDtypeStruct((e, d), jnp.int32),
            jax.ShapeDtypeStruct((e, d), jnp.int32),
        ],
        mesh=mesh,
        scratch_types=[
            pltpu.VMEM((CH,), jnp.int32),
            pltpu.VMEM((CH,), jnp.int32),
            pltpu.VMEM((CH, d), jnp.int32),
            pltpu.VMEM((CH, d), jnp.int32),
            pltpu.SemaphoreType.DMA,
            pltpu.SemaphoreType.DMA,
        ],
    )
    def gather_k(p_hbm, q_hbm, src_hbm, dst_hbm, z1_hbm, z2_hbm,
                 sidx, didx, pbuf, qbuf, sem_p, sem_q):
        wid = lax.axis_index("s") * NC + lax.axis_index("c")
        base0 = wid * epw

        def chunk_body(i, carry):
            base = base0 + i * CH
            pltpu.sync_copy(src_hbm.at[pl.ds(base, CH)], sidx)
            pltpu.sync_copy(dst_hbm.at[pl.ds(base, CH)], didx)
            cp_p = pltpu.async_copy(p_hbm.at[sidx], pbuf, sem_p)
            cp_q = pltpu.async_copy(q_hbm.at[didx], qbuf, sem_q)
            cp_p.wait()
            cp_q.wait()
            pltpu.sync_copy(pbuf, z1_hbm.at[pl.ds(base, CH)])
            pltpu.sync_copy(qbuf, z2_hbm.at[pl.ds(base, CH)])
            return carry

        lax.fori_loop(0, nch, chunk_body, 0)

    return gather_k(p, q, src, dst)


def _edge_mm_body(z1_ref, z2_ref, ef_ref, wet_ref, bgc_ref, m_ref):
    d = m_ref.shape[1]
    r = jnp.dot(ef_ref[...], wet_ref[...],
                preferred_element_type=jnp.float32) + bgc_ref[...]
    g1, c1 = _unpack_bf16(z1_ref[...])
    g2, c2 = _unpack_bf16(z2_ref[...])
    zg = g1 + g2 + r[:, :d]
    zc = c1 + c2 + r[:, d:]
    m_ref[...] = jax.nn.sigmoid(zg) * jax.nn.softplus(zc)


def _edge_mm(z1, z2, ef, wet, bgc, block_rows):
    e, d = z1.shape
    de = ef.shape[1]
    d2 = 2 * d
    grid = (e // block_rows,)
    return pl.pallas_call(
        _edge_mm_body,
        grid=grid,
        in_specs=[
            pl.BlockSpec((block_rows, d), lambda i: (i, 0)),
            pl.BlockSpec((block_rows, d), lambda i: (i, 0)),
            pl.BlockSpec((block_rows, de), lambda i: (i, 0)),
            pl.BlockSpec((de, d2), lambda i: (0, 0)),
            pl.BlockSpec((1, d2), lambda i: (0, 0)),
        ],
        out_specs=pl.BlockSpec((block_rows, d), lambda i: (i, 0)),
        out_shape=jax.ShapeDtypeStruct((e, d), jnp.float32),
    )(z1, z2, ef, wet, bgc)


def _sc_scatter(m, src, n):
    e, d = m.shape
    epw = e // NW
    nch = epw // CH
    fc = 128
    npt = ((n + NS * fc - 1) // (NS * fc)) * fc
    n_pad = npt * NS
    nfc = npt // fc
    mesh = plsc.VectorSubcoreMesh(core_axis_name="c", subcore_axis_name="s")

    @functools.partial(
        pl.kernel,
        out_type=jax.ShapeDtypeStruct((NC, n_pad, d), jnp.float32),
        mesh=mesh,
        scratch_types=[
            pltpu.VMEM((CH,), jnp.int32),
            pltpu.VMEM((CH, d), jnp.float32),
            pltpu.VMEM((fc, d), jnp.float32),
            pltpu.VMEM_SHARED((n_pad, d), jnp.float32),
        ],
    )
    def scatter_k(m_hbm, src_hbm, out_hbm, idxv, mbuf, stg, acc_sh):
        cid = lax.axis_index("c")
        sid = lax.axis_index("s")
        wid = sid * NC + cid
        base0 = wid * epw

        def zero_body(r, c):
            for k in range(d // NL):
                stg[r, pl.ds(k * NL, NL)] = jnp.zeros((NL,), jnp.float32)
            return c

        lax.fori_loop(0, fc, zero_body, 0)
        for j in range(nfc):
            pltpu.sync_copy(stg, acc_sh.at[pl.ds(sid * npt + j * fc, fc)])
        plsc.subcore_barrier()

        def chunk_body(i, c):
            base = base0 + i * CH
            pltpu.sync_copy(src_hbm.at[pl.ds(base, CH)], idxv)
            pltpu.sync_copy(m_hbm.at[pl.ds(base, CH)], mbuf)
            pltpu.sync_copy(mbuf, acc_sh.at[idxv], add=True)
            return c

        lax.fori_loop(0, nch, chunk_body, 0)
        plsc.subcore_barrier()
        for j in range(nfc):
            r0 = sid * npt + j * fc
            pltpu.sync_copy(acc_sh.at[pl.ds(r0, fc)], stg)
            pltpu.sync_copy(stg, out_hbm.at[cid, pl.ds(r0, fc)])

    return scatter_k(m, src)


def _sum_parts_body(*refs):
    o_ref = refs[-1]
    acc = refs[0][0] + refs[0][1]
    for pr in refs[1:-1]:
        acc = acc + pr[0] + pr[1]
    o_ref[...] = acc


def _sum_parts(parts, rblk=1280):
    ncores, n_pad, d = parts[0].shape
    grid = (n_pad // rblk,)
    spec = pl.BlockSpec((ncores, rblk, d), lambda i: (0, i, 0))
    return pl.pallas_call(
        _sum_parts_body,
        grid=grid,
        in_specs=[spec] * len(parts),
        out_specs=pl.BlockSpec((rblk, d), lambda i: (i, 0)),
        out_shape=jax.ShapeDtypeStruct((n_pad, d), jnp.float32),
    )(*parts)


def _bn_body(agg_ref, h_ref, gamma_ref, beta_ref, o_ref):
    agg = agg_ref[...]
    n = agg.shape[0]
    mean = jnp.sum(agg, axis=0, keepdims=True) / n
    cen = agg - mean
    var = jnp.sum(cen * cen, axis=0, keepdims=True) / n
    xb = cen * lax.rsqrt(var + 1e-5) * gamma_ref[...] + beta_ref[...]
    o_ref[...] = jax.nn.softplus(h_ref[...] + xb)


def _bn_final(agg, h, gamma, beta):
    n, d = h.shape
    return pl.pallas_call(
        _bn_body,
        in_specs=[
            pl.BlockSpec((n, d), lambda: (0, 0)),
            pl.BlockSpec((n, d), lambda: (0, 0)),
            pl.BlockSpec((1, d), lambda: (0, 0)),
            pl.BlockSpec((1, d), lambda: (0, 0)),
        ],
        out_specs=pl.BlockSpec((n, d), lambda: (0, 0)),
        out_shape=jax.ShapeDtypeStruct((n, d), jnp.float32),
    )(agg, h, gamma, beta)


def kernel(h, edge_index, edge_feat, Wg, bg, Wc, bc, gamma, beta):
    n, d = h.shape
    e = edge_index.shape[1]
    src = edge_index[0]
    dst = edge_index[1]

    wsg = Wg[:, :d].T
    wsc = Wc[:, :d].T
    wdg = Wg[:, d:2 * d].T
    wdc = Wc[:, d:2 * d].T
    wet = jnp.concatenate([Wg[:, 2 * d:], Wc[:, 2 * d:]], axis=0).T
    bgc = jnp.concatenate([bg, bc]).reshape(1, 2 * d)

    p, q = _node_mm(h, wsg, wsc, wdg, wdc, block_rows=2000)

    K = 5
    es = e // K
    parts = []
    for k in range(K):
        sl = slice(k * es, (k + 1) * es)
        z1, z2 = _sc_gather(p, q, src[sl], dst[sl])
        m = _edge_mm(z1, z2, edge_feat[sl], wet, bgc, block_rows=3200)
        parts.append(_sc_scatter(m, src[sl], n))
    agg = _sum_parts(parts)[:n]
    return _bn_final(agg, h, gamma.reshape(1, d), beta.reshape(1, d))


# speedup vs baseline: 1.3336x; 1.0067x over previous
import functools

import jax
import jax.numpy as jnp
from jax import lax
from jax.experimental import pallas as pl
from jax.experimental.pallas import tpu as pltpu
from jax.experimental.pallas import tpu_sc as plsc

NC = 2
NS = 16
NL = 16
NW = NC * NS

CH = 80


def _pack_bf16(a, b):
    ai = lax.bitcast_convert_type(a, jnp.int32)
    bi = lax.bitcast_convert_type(b, jnp.int32)
    ar = ai + jnp.int32(0x7FFF) + ((ai >> 16) & 1)
    br = bi + jnp.int32(0x7FFF) + ((bi >> 16) & 1)
    lo = (ar >> 16) & jnp.int32(0xFFFF)
    hi = br & jnp.int32(-0x10000)
    return lo | hi


def _unpack_bf16(x):
    lo = lax.bitcast_convert_type(x << 16, jnp.float32)
    hi = lax.bitcast_convert_type(x & jnp.int32(-0x10000), jnp.float32)
    return lo, hi


def _node_mm_body(h_ref, wsg_ref, wsc_ref, wdg_ref, wdc_ref, p_ref, q_ref):
    hb = h_ref[...]
    pg = jnp.dot(hb, wsg_ref[...], preferred_element_type=jnp.float32)
    pc = jnp.dot(hb, wsc_ref[...], preferred_element_type=jnp.float32)
    qg = jnp.dot(hb, wdg_ref[...], preferred_element_type=jnp.float32)
    qc = jnp.dot(hb, wdc_ref[...], preferred_element_type=jnp.float32)
    p_ref[...] = _pack_bf16(pg, pc)
    q_ref[...] = _pack_bf16(qg, qc)


def _node_mm(h, wsg, wsc, wdg, wdc, block_rows):
    n, d = h.shape
    grid = (n // block_rows,)
    wspec = pl.BlockSpec((d, d), lambda i: (0, 0))
    return pl.pallas_call(
        _node_mm_body,
        grid=grid,
        in_specs=[
            pl.BlockSpec((block_rows, d), lambda i: (i, 0)),
            wspec, wspec, wspec, wspec,
        ],
        out_specs=[
            pl.BlockSpec((block_rows, d), lambda i: (i, 0)),
            pl.BlockSpec((block_rows, d), lambda i: (i, 0)),
        ],
        out_shape=[
            jax.ShapeDtypeStruct((n, d), jnp.int32),
            jax.ShapeDtypeStruct((n, d), jnp.int32),
        ],
    )(h, wsg, wsc, wdg, wdc)


def _sc_gather(p, q, src, dst):
    n, d = p.shape
    e = src.shape[0]
    epw = e // NW
    nch = epw // CH
    mesh = plsc.VectorSubcoreMesh(core_axis_name="c", subcore_axis_name="s")

    rps = 640
    rem = n - (NS - 1) * rps

    @functools.partial(
        pl.kernel,
        out_type=[
            jax.ShapeDtypeStruct((e, d), jnp.int32),
            jax.ShapeDtypeStruct((e, d), jnp.int32),
        ],
        mesh=mesh,
        scratch_types=[
            pltpu.VMEM((CH,), jnp.int32),
            pltpu.VMEM((CH,), jnp.int32),
            pltpu.VMEM((CH, d), jnp.int32),
            pltpu.VMEM((CH, d), jnp.int32),
            pltpu.VMEM_SHARED((n, d), jnp.int32),
            pltpu.SemaphoreType.DMA,
            pltpu.SemaphoreType.DMA,
        ],
    )
    def gather_k(p_hbm, q_hbm, src_hbm, dst_hbm, z1_hbm, z2_hbm,
                 sidx, didx, pbuf, qbuf, p_sh, sem_p, sem_q):
        sid = lax.axis_index("s")
        wid = sid * NC + lax.axis_index("c")
        base0 = wid * epw

        r0 = sid * rps

        @pl.when(sid < NS - 1)
        def _():
            pltpu.sync_copy(p_hbm.at[pl.ds(r0, rps)], p_sh.at[pl.ds(r0, rps)])

        @pl.when(sid == NS - 1)
        def _():
            pltpu.sync_copy(p_hbm.at[pl.ds((NS - 1) * rps, rem)],
                            p_sh.at[pl.ds((NS - 1) * rps, rem)])

        plsc.subcore_barrier()

        def chunk_body(i, carry):
            base = base0 + i * CH
            pltpu.sync_copy(src_hbm.at[pl.ds(base, CH)], sidx)
            pltpu.sync_copy(dst_hbm.at[pl.ds(base, CH)], didx)
            cp_p = pltpu.async_copy(p_sh.at[sidx], pbuf, sem_p)
            cp_q = pltpu.async_copy(q_hbm.at[didx], qbuf, sem_q)
            cp_p.wait()
            cp_q.wait()
            pltpu.sync_copy(pbuf, z1_hbm.at[pl.ds(base, CH)])
            pltpu.sync_copy(qbuf, z2_hbm.at[pl.ds(base, CH)])
            return carry

        lax.fori_loop(0, nch, chunk_body, 0)

    return gather_k(p, q, src, dst)


def _edge_mm_body(z1_ref, z2_ref, ef_ref, wet_ref, bgc_ref, m_ref):
    d = m_ref.shape[1]
    r = jnp.dot(ef_ref[...], wet_ref[...],
                preferred_element_type=jnp.float32) + bgc_ref[...]
    g1, c1 = _unpack_bf16(z1_ref[...])
    g2, c2 = _unpack_bf16(z2_ref[...])
    zg = g1 + g2 + r[:, :d]
    zc = c1 + c2 + r[:, d:]
    m_ref[...] = jax.nn.sigmoid(zg) * jax.nn.softplus(zc)


def _edge_mm(z1, z2, ef, wet, bgc, block_rows):
    e, d = z1.shape
    de = ef.shape[1]
    d2 = 2 * d
    grid = (e // block_rows,)
    return pl.pallas_call(
        _edge_mm_body,
        grid=grid,
        in_specs=[
            pl.BlockSpec((block_rows, d), lambda i: (i, 0)),
            pl.BlockSpec((block_rows, d), lambda i: (i, 0)),
            pl.BlockSpec((block_rows, de), lambda i: (i, 0)),
            pl.BlockSpec((de, d2), lambda i: (0, 0)),
            pl.BlockSpec((1, d2), lambda i: (0, 0)),
        ],
        out_specs=pl.BlockSpec((block_rows, d), lambda i: (i, 0)),
        out_shape=jax.ShapeDtypeStruct((e, d), jnp.float32),
    )(z1, z2, ef, wet, bgc)


def _sc_scatter(m, src, n):
    e, d = m.shape
    epw = e // NW
    nch = epw // CH
    fc = 128
    npt = ((n + NS * fc - 1) // (NS * fc)) * fc
    n_pad = npt * NS
    nfc = npt // fc
    mesh = plsc.VectorSubcoreMesh(core_axis_name="c", subcore_axis_name="s")

    @functools.partial(
        pl.kernel,
        out_type=jax.ShapeDtypeStruct((NC, n_pad, d), jnp.float32),
        mesh=mesh,
        scratch_types=[
            pltpu.VMEM((CH,), jnp.int32),
            pltpu.VMEM((CH, d), jnp.float32),
            pltpu.VMEM((fc, d), jnp.float32),
            pltpu.VMEM_SHARED((n_pad, d), jnp.float32),
        ],
    )
    def scatter_k(m_hbm, src_hbm, out_hbm, idxv, mbuf, stg, acc_sh):
        cid = lax.axis_index("c")
        sid = lax.axis_index("s")
        wid = sid * NC + cid
        base0 = wid * epw

        def zero_body(r, c):
            for k in range(d // NL):
                stg[r, pl.ds(k * NL, NL)] = jnp.zeros((NL,), jnp.float32)
            return c

        lax.fori_loop(0, fc, zero_body, 0)
        for j in range(nfc):
            pltpu.sync_copy(stg, acc_sh.at[pl.ds(sid * npt + j * fc, fc)])
        plsc.subcore_barrier()

        def chunk_body(i, c):
            base = base0 + i * CH
            pltpu.sync_copy(src_hbm.at[pl.ds(base, CH)], idxv)
            pltpu.sync_copy(m_hbm.at[pl.ds(base, CH)], mbuf)
            pltpu.sync_copy(mbuf, acc_sh.at[idxv], add=True)
            return c

        lax.fori_loop(0, nch, chunk_body, 0)
        plsc.subcore_barrier()
        for j in range(nfc):
            r0 = sid * npt + j * fc
            pltpu.sync_copy(acc_sh.at[pl.ds(r0, fc)], stg)
            pltpu.sync_copy(stg, out_hbm.at[cid, pl.ds(r0, fc)])

    return scatter_k(m, src)


def _sum_parts_body(*refs):
    o_ref = refs[-1]
    acc = refs[0][0] + refs[0][1]
    for pr in refs[1:-1]:
        acc = acc + pr[0] + pr[1]
    o_ref[...] = acc


def _sum_parts(parts, rblk=1280):
    ncores, n_pad, d = parts[0].shape
    grid = (n_pad // rblk,)
    spec = pl.BlockSpec((ncores, rblk, d), lambda i: (0, i, 0))
    return pl.pallas_call(
        _sum_parts_body,
        grid=grid,
        in_specs=[spec] * len(parts),
        out_specs=pl.BlockSpec((rblk, d), lambda i: (i, 0)),
        out_shape=jax.ShapeDtypeStruct((n_pad, d), jnp.float32),
    )(*parts)


def _bn_body(agg_ref, h_ref, gamma_ref, beta_ref, o_ref):
    agg = agg_ref[...]
    n = agg.shape[0]
    mean = jnp.sum(agg, axis=0, keepdims=True) / n
    cen = agg - mean
    var = jnp.sum(cen * cen, axis=0, keepdims=True) / n
    xb = cen * lax.rsqrt(var + 1e-5) * gamma_ref[...] + beta_ref[...]
    o_ref[...] = jax.nn.softplus(h_ref[...] + xb)


def _bn_final(agg, h, gamma, beta):
    n, d = h.shape
    return pl.pallas_call(
        _bn_body,
        in_specs=[
            pl.BlockSpec((n, d), lambda: (0, 0)),
            pl.BlockSpec((n, d), lambda: (0, 0)),
            pl.BlockSpec((1, d), lambda: (0, 0)),
            pl.BlockSpec((1, d), lambda: (0, 0)),
        ],
        out_specs=pl.BlockSpec((n, d), lambda: (0, 0)),
        out_shape=jax.ShapeDtypeStruct((n, d), jnp.float32),
    )(agg, h, gamma, beta)


def kernel(h, edge_index, edge_feat, Wg, bg, Wc, bc, gamma, beta):
    n, d = h.shape
    e = edge_index.shape[1]
    src = edge_index[0]
    dst = edge_index[1]

    wsg = Wg[:, :d].T
    wsc = Wc[:, :d].T
    wdg = Wg[:, d:2 * d].T
    wdc = Wc[:, d:2 * d].T
    wet = jnp.concatenate([Wg[:, 2 * d:], Wc[:, 2 * d:]], axis=0).T
    bgc = jnp.concatenate([bg, bc]).reshape(1, 2 * d)

    p, q = _node_mm(h, wsg, wsc, wdg, wdc, block_rows=2000)

    K = 5
    es = e // K
    parts = []
    for k in range(K):
        sl = slice(k * es, (k + 1) * es)
        z1, z2 = _sc_gather(p, q, src[sl], dst[sl])
        m = _edge_mm(z1, z2, edge_feat[sl], wet, bgc, block_rows=3200)
        parts.append(_sc_scatter(m, src[sl], n))
    agg = _sum_parts(parts)[:n]
    return _bn_final(agg, h, gamma.reshape(1, d), beta.reshape(1, d))
